# Initial kernel scaffold; baseline (speedup 1.0000x reference)
import jax
import jax.numpy as jnp
from jax.experimental import pallas as pl


def kernel(Q, A, AT, b, c, x, y, Iy, il, iu, l, u):
    raise NotImplementedError("write your pallas kernel here")


# speedup vs baseline: 1.7374x; 1.7374x over previous
import jax
import jax.numpy as jnp
from jax.experimental import pallas as pl

_B = 512


def _matvec_body(xT_ref, A_ref, Q_ref, yblk_ref, Ax_ref, Qx_ref, ATy_ref):
    i = pl.program_id(0)
    A = A_ref[...]
    Q = Q_ref[...]
    xT = xT_ref[...]
    yb = yblk_ref[...]
    dn_row = (((1,), (1,)), ((), ()))
    Ax_ref[...] = jax.lax.dot_general(xT, A, dn_row,
                                      preferred_element_type=jnp.float32)
    Qx_ref[...] = jax.lax.dot_general(xT, Q, dn_row,
                                      preferred_element_type=jnp.float32)
    contrib = jax.lax.dot_general(yb, A, (((1,), (0,)), ((), ())),
                                  preferred_element_type=jnp.float32)

    @pl.when(i == 0)
    def _init():
        ATy_ref[...] = contrib

    @pl.when(i > 0)
    def _acc():
        ATy_ref[...] = ATy_ref[...] + contrib


def _finalize_body(Ax_ref, Qx_ref, ATy_ref, b_ref, c_ref, x_ref, y_ref,
                   Iy_ref, il_ref, iu_ref, l_ref, u_ref,
                   res_ref, t1_ref, t2_ref, t3_ref):
    relu = lambda v: jnp.maximum(v, 0.0)
    Ax = Ax_ref[...]
    Qx = Qx_ref[...]
    ATy = ATy_ref[...]
    b = b_ref[...]
    c = c_ref[...]
    x = x_ref[...]
    y = y_ref[...]
    Iy = Iy_ref[...]
    il = il_ref[...]
    iu = iu_ref[...]
    l = l_ref[...]
    u = u_ref[...]
    cons = Ax - b
    cons = cons + relu(-cons) * Iy
    var = relu(l - x) * il + relu(x - u) * iu
    part2 = jnp.maximum(jnp.max(jnp.abs(var)), jnp.max(jnp.abs(cons)))
    t1 = part2 / (1.0 + jnp.max(jnp.abs(b)))
    quad = jnp.sum(x * Qx)
    lin = jnp.sum(c * x)
    vio = jnp.sum(b * y)
    pg_g = c - ATy + Qx
    RC = relu(pg_g) * il - relu(-pg_g) * iu
    tm = jnp.where(RC > 0, l, u)
    rc = jnp.sum(RC * tm)
    top_g = jnp.abs(quad + lin - vio - rc)
    bot_g = 1.0 + jnp.maximum(jnp.abs(vio - 0.5 * quad),
                              jnp.abs(0.5 * quad + lin))
    t3 = top_g / bot_g
    pg = c + ATy + Qx
    RCV = pg - relu(pg) * il - relu(-pg) * iu
    DR = relu(-y) * Iy
    t2 = jnp.maximum(jnp.max(jnp.abs(RCV)), jnp.max(jnp.abs(DR))) / \
        (1.0 + jnp.max(jnp.abs(c)))
    res_ref[...] = jnp.reshape(t1 + t2 + t3, (1, 1))
    t1_ref[...] = jnp.reshape(t1, (1, 1))
    t2_ref[...] = jnp.reshape(t2, (1, 1))
    t3_ref[...] = jnp.reshape(t3, (1, 1))


def kernel(Q, A, AT, b, c, x, y, Iy, il, iu, l, u):
    del AT
    m, n = A.shape
    nb = m // _B
    xT = x.reshape(1, n)
    yT = y.reshape(1, m)

    Ax, Qx, ATy = pl.pallas_call(
        _matvec_body,
        grid=(nb,),
        in_specs=[
            pl.BlockSpec((1, n), lambda i: (0, 0)),
            pl.BlockSpec((_B, n), lambda i: (i, 0)),
            pl.BlockSpec((_B, n), lambda i: (i, 0)),
            pl.BlockSpec((1, _B), lambda i: (0, i)),
        ],
        out_specs=[
            pl.BlockSpec((1, _B), lambda i: (0, i)),
            pl.BlockSpec((1, _B), lambda i: (0, i)),
            pl.BlockSpec((1, n), lambda i: (0, 0)),
        ],
        out_shape=[
            jax.ShapeDtypeStruct((1, m), jnp.float32),
            jax.ShapeDtypeStruct((1, n), jnp.float32),
            jax.ShapeDtypeStruct((1, n), jnp.float32),
        ],
    )(xT, A, Q, yT)

    shp = (32, n // 32)
    sd = jax.ShapeDtypeStruct((1, 1), jnp.float32)
    res, t1, t2, t3 = pl.pallas_call(
        _finalize_body,
        out_shape=[sd, sd, sd, sd],
    )(Ax.reshape(shp), Qx.reshape(shp), ATy.reshape(shp),
      b.reshape(shp), c.reshape(shp), x.reshape(shp), y.reshape(shp),
      Iy.reshape(shp), il.reshape(shp), iu.reshape(shp),
      l.reshape(shp), u.reshape(shp))
    return (res, t1.reshape(()), t2.reshape(()), t3)


# speedup vs baseline: 1.7630x; 1.0147x over previous
import jax
import jax.numpy as jnp
from jax.experimental import pallas as pl

_B = 256


def _matvec_body(xT_ref, A_ref, Q_ref, yblk_ref, Ax_ref, Qx_ref, ATy_ref):
    i = pl.program_id(0)
    A = A_ref[...]
    Q = Q_ref[...]
    xT = xT_ref[...]
    yb = yblk_ref[...]
    dn_row = (((1,), (1,)), ((), ()))
    Ax_ref[...] = jax.lax.dot_general(xT, A, dn_row,
                                      preferred_element_type=jnp.float32)
    Qx_ref[...] = jax.lax.dot_general(xT, Q, dn_row,
                                      preferred_element_type=jnp.float32)
    contrib = jax.lax.dot_general(yb, A, (((1,), (0,)), ((), ())),
                                  preferred_element_type=jnp.float32)

    @pl.when(i == 0)
    def _init():
        ATy_ref[...] = contrib

    @pl.when(i > 0)
    def _acc():
        ATy_ref[...] = ATy_ref[...] + contrib


def _finalize_body(Ax_ref, Qx_ref, ATy_ref, b_ref, c_ref, x_ref, y_ref,
                   Iy_ref, il_ref, iu_ref, l_ref, u_ref,
                   res_ref, t1_ref, t2_ref, t3_ref):
    relu = lambda v: jnp.maximum(v, 0.0)
    Ax = Ax_ref[...]
    Qx = Qx_ref[...]
    ATy = ATy_ref[...]
    b = b_ref[...]
    c = c_ref[...]
    x = x_ref[...]
    y = y_ref[...]
    Iy = Iy_ref[...]
    il = il_ref[...]
    iu = iu_ref[...]
    l = l_ref[...]
    u = u_ref[...]
    cons = Ax - b
    cons = cons + relu(-cons) * Iy
    var = relu(l - x) * il + relu(x - u) * iu
    part2 = jnp.maximum(jnp.max(jnp.abs(var)), jnp.max(jnp.abs(cons)))
    t1 = part2 / (1.0 + jnp.max(jnp.abs(b)))
    quad = jnp.sum(x * Qx)
    lin = jnp.sum(c * x)
    vio = jnp.sum(b * y)
    pg_g = c - ATy + Qx
    RC = relu(pg_g) * il - relu(-pg_g) * iu
    tm = jnp.where(RC > 0, l, u)
    rc = jnp.sum(RC * tm)
    top_g = jnp.abs(quad + lin - vio - rc)
    bot_g = 1.0 + jnp.maximum(jnp.abs(vio - 0.5 * quad),
                              jnp.abs(0.5 * quad + lin))
    t3 = top_g / bot_g
    pg = c + ATy + Qx
    RCV = pg - relu(pg) * il - relu(-pg) * iu
    DR = relu(-y) * Iy
    t2 = jnp.maximum(jnp.max(jnp.abs(RCV)), jnp.max(jnp.abs(DR))) / \
        (1.0 + jnp.max(jnp.abs(c)))
    res_ref[...] = jnp.reshape(t1 + t2 + t3, (1, 1))
    t1_ref[...] = jnp.reshape(t1, (1, 1))
    t2_ref[...] = jnp.reshape(t2, (1, 1))
    t3_ref[...] = jnp.reshape(t3, (1, 1))


def kernel(Q, A, AT, b, c, x, y, Iy, il, iu, l, u):
    del AT
    m, n = A.shape
    nb = m // _B
    xT = x.reshape(1, n)
    yT = y.reshape(1, m)

    Ax, Qx, ATy = pl.pallas_call(
        _matvec_body,
        grid=(nb,),
        in_specs=[
            pl.BlockSpec((1, n), lambda i: (0, 0)),
            pl.BlockSpec((_B, n), lambda i: (i, 0)),
            pl.BlockSpec((_B, n), lambda i: (i, 0)),
            pl.BlockSpec((1, _B), lambda i: (0, i)),
        ],
        out_specs=[
            pl.BlockSpec((1, _B), lambda i: (0, i)),
            pl.BlockSpec((1, _B), lambda i: (0, i)),
            pl.BlockSpec((1, n), lambda i: (0, 0)),
        ],
        out_shape=[
            jax.ShapeDtypeStruct((1, m), jnp.float32),
            jax.ShapeDtypeStruct((1, n), jnp.float32),
            jax.ShapeDtypeStruct((1, n), jnp.float32),
        ],
    )(xT, A, Q, yT)

    shp = (32, n // 32)
    sd = jax.ShapeDtypeStruct((1, 1), jnp.float32)
    res, t1, t2, t3 = pl.pallas_call(
        _finalize_body,
        out_shape=[sd, sd, sd, sd],
    )(Ax.reshape(shp), Qx.reshape(shp), ATy.reshape(shp),
      b.reshape(shp), c.reshape(shp), x.reshape(shp), y.reshape(shp),
      Iy.reshape(shp), il.reshape(shp), iu.reshape(shp),
      l.reshape(shp), u.reshape(shp))
    return (res, t1.reshape(()), t2.reshape(()), t3)
